# baseline (speedup 1.0000x reference)
import functools

import jax
import jax.numpy as jnp
from jax import lax
from jax.experimental import pallas as pl
from jax.experimental.pallas import tpu as pltpu
from jax.experimental.pallas import tpu_sc as plsc

NUM_EXPERTS = 64
HIDDEN = 1024
INTER = 768
TOKENS = 2048
TOP_K = 2
PAIRS = TOKENS * TOP_K
ALIGN = 8
RT = 64
PBUF = 4608

SC_CORES = 2
SC_SUBCORES = 16
SC_WORKERS = SC_CORES * SC_SUBCORES

A_PER_W = PAIRS // SC_WORKERS
A_CHUNK = 32
A_NCHUNK = A_PER_W // A_CHUNK

C_PER_W = TOKENS // SC_WORKERS
C_CHUNK = 16
C_NCHUNK = C_PER_W // C_CHUNK

def _worker_id():
    return lax.axis_index("s") * SC_CORES + lax.axis_index("c")


@functools.lru_cache(maxsize=None)
def _sc_kernels():
    mesh = plsc.VectorSubcoreMesh(core_axis_name="c", subcore_axis_name="s")

    @functools.partial(
        pl.kernel,
        mesh=mesh,
        out_type=jax.ShapeDtypeStruct((PBUF, HIDDEN), jnp.float32),
        scratch_types=[
            pltpu.VMEM((A_NCHUNK, A_CHUNK), jnp.int32),
            pltpu.VMEM((A_NCHUNK, A_CHUNK), jnp.int32),
            pltpu.VMEM((A_CHUNK, HIDDEN), jnp.float32),
            pltpu.VMEM((A_CHUNK, HIDDEN), jnp.float32),
            pltpu.SemaphoreType.DMA,
            pltpu.SemaphoreType.DMA,
            pltpu.SemaphoreType.DMA,
            pltpu.SemaphoreType.DMA,
        ],
    )
    def sc_dispatch(hidden_hbm, tok3_hbm, pp3_hbm, out_hbm, tok_v, pp_v,
                    row0_v, row1_v, sem_g0, sem_g1, sem_s0, sem_s1):
        wid = _worker_id()
        pltpu.sync_copy(tok3_hbm.at[wid], tok_v)
        pltpu.sync_copy(pp3_hbm.at[wid], pp_v)
        bufs = (row0_v, row1_v)
        gsems = (sem_g0, sem_g1)
        ssems = (sem_s0, sem_s1)
        gathers = [None] * A_NCHUNK
        stores = [None] * A_NCHUNK
        for c in range(A_NCHUNK):
            if c >= 2:
                stores[c - 2].wait()
            gathers[c] = pltpu.async_copy(
                hidden_hbm.at[tok_v.at[c]], bufs[c % 2], gsems[c % 2])
            if c >= 1:
                gathers[c - 1].wait()
                stores[c - 1] = pltpu.async_copy(
                    bufs[(c - 1) % 2], out_hbm.at[pp_v.at[c - 1]],
                    ssems[(c - 1) % 2])
        gathers[A_NCHUNK - 1].wait()
        stores[A_NCHUNK - 1] = pltpu.async_copy(
            bufs[(A_NCHUNK - 1) % 2], out_hbm.at[pp_v.at[A_NCHUNK - 1]],
            ssems[(A_NCHUNK - 1) % 2])
        stores[A_NCHUNK - 2].wait()
        stores[A_NCHUNK - 1].wait()

    @functools.partial(
        pl.kernel,
        mesh=mesh,
        out_type=jax.ShapeDtypeStruct((TOKENS, HIDDEN), jnp.float32),
        scratch_types=[
            pltpu.VMEM((C_PER_W,), jnp.int32),
            pltpu.VMEM((C_PER_W,), jnp.int32),
            pltpu.VMEM((C_PER_W,), jnp.float32),
            pltpu.VMEM((C_PER_W,), jnp.float32),
            pltpu.VMEM((C_CHUNK, HIDDEN), jnp.float32),
            pltpu.VMEM((C_CHUNK, HIDDEN), jnp.float32),
            pltpu.VMEM((C_CHUNK, HIDDEN), jnp.float32),
            pltpu.VMEM((C_CHUNK, HIDDEN), jnp.float32),
            pltpu.VMEM((C_CHUNK, HIDDEN), jnp.float32),
            pltpu.VMEM((C_CHUNK, HIDDEN), jnp.float32),
            pltpu.SemaphoreType.DMA,
            pltpu.SemaphoreType.DMA,
            pltpu.SemaphoreType.DMA,
            pltpu.SemaphoreType.DMA,
            pltpu.SemaphoreType.DMA,
            pltpu.SemaphoreType.DMA,
            pltpu.SemaphoreType.DMA,
            pltpu.SemaphoreType.DMA,
            pltpu.SemaphoreType.DMA,
        ],
    )
    def sc_combine(ypad_hbm, i0_hbm, i1_hbm, w0_hbm, w1_hbm, out_hbm,
                   ia_v, ib_v, wa_v, wb_v, a0_v, a1_v, a2_v, b0_v, b1_v, b2_v,
                   sem_ga0, sem_ga1, sem_ga2, sem_gb0, sem_gb1, sem_gb2,
                   sem_s0, sem_s1, sem_s2):
        wid = _worker_id()
        base = wid * C_PER_W
        pltpu.sync_copy(i0_hbm.at[pl.ds(base, C_PER_W)], ia_v)
        pltpu.sync_copy(i1_hbm.at[pl.ds(base, C_PER_W)], ib_v)
        pltpu.sync_copy(w0_hbm.at[pl.ds(base, C_PER_W)], wa_v)
        pltpu.sync_copy(w1_hbm.at[pl.ds(base, C_PER_W)], wb_v)
        abufs = (a0_v, a1_v, a2_v)
        bbufs = (b0_v, b1_v, b2_v)
        ga_sems = (sem_ga0, sem_ga1, sem_ga2)
        gb_sems = (sem_gb0, sem_gb1, sem_gb2)
        s_sems = (sem_s0, sem_s1, sem_s2)
        ga = [None] * C_NCHUNK
        gb = [None] * C_NCHUNK
        st = [None] * C_NCHUNK

        def fire(c):
            ga[c] = pltpu.async_copy(
                ypad_hbm.at[ia_v.at[pl.ds(c * C_CHUNK, C_CHUNK)]],
                abufs[c % 3], ga_sems[c % 3])
            gb[c] = pltpu.async_copy(
                ypad_hbm.at[ib_v.at[pl.ds(c * C_CHUNK, C_CHUNK)]],
                bbufs[c % 3], gb_sems[c % 3])

        fire(0)
        fire(1)
        fire(2)
        for c in range(C_NCHUNK):
            ga[c].wait()
            gb[c].wait()
            a_v = abufs[c % 3]
            b_v = bbufs[c % 3]
            for r in range(C_CHUNK):
                g = c * C_CHUNK + r
                wa = wa_v[pl.ds((g // 16) * 16, 16)][g % 16]
                wb = wb_v[pl.ds((g // 16) * 16, 16)][g % 16]

                def col_body(j, _, r=r, wa=wa, wb=wb, a_v=a_v, b_v=b_v):
                    sl = pl.ds(j * 16, 16)
                    a_v[r, sl] = a_v[r, sl] * wa + b_v[r, sl] * wb
                    return 0
                lax.fori_loop(0, HIDDEN // 16, col_body, 0, unroll=8)

            st[c] = pltpu.async_copy(
                a_v, out_hbm.at[pl.ds(base + c * C_CHUNK, C_CHUNK)],
                s_sems[c % 3])
            if c + 3 < C_NCHUNK:
                st[c].wait()
                fire(c + 3)
        for c in range(max(0, C_NCHUNK - 3), C_NCHUNK):
            st[c].wait()

    return sc_dispatch, sc_combine


def _tc_moe_body(poff_ref, x_ref, gu_ref, dn_ref, y_ref):
    e = pl.program_id(0)
    start = poff_ref[e]
    end = poff_ref[e + 1]
    ntiles = (end - start + RT - 1) // RT
    dn = dn_ref[0]

    def tile(i, _):
        r0 = pl.multiple_of(start + i * RT, ALIGN)
        x = x_ref[pl.ds(r0, RT), :]
        g = lax.dot_general(x, gu_ref[0, :INTER, :],
                            (((1,), (1,)), ((), ())),
                            preferred_element_type=jnp.float32)
        u = lax.dot_general(x, gu_ref[0, INTER:, :],
                            (((1,), (1,)), ((), ())),
                            preferred_element_type=jnp.float32)
        h = g * jax.nn.sigmoid(g) * u
        y = lax.dot_general(h, dn, (((1,), (1,)), ((), ())),
                            preferred_element_type=jnp.float32)
        y_ref[pl.ds(r0, RT), :] = y
        return 0

    lax.fori_loop(0, ntiles, tile, 0)


def _tc_moe(x_pad, gate_up_proj, down_proj, poff, interpret=False):
    grid_spec = pltpu.PrefetchScalarGridSpec(
        num_scalar_prefetch=1,
        grid=(NUM_EXPERTS,),
        in_specs=[
            pl.BlockSpec((PBUF, HIDDEN), lambda e, poff: (0, 0)),
            pl.BlockSpec((1, 2 * INTER, HIDDEN), lambda e, poff: (e, 0, 0)),
            pl.BlockSpec((1, HIDDEN, INTER), lambda e, poff: (e, 0, 0)),
        ],
        out_specs=pl.BlockSpec((PBUF, HIDDEN), lambda e, poff: (0, 0)),
    )
    return pl.pallas_call(
        _tc_moe_body,
        grid_spec=grid_spec,
        out_shape=jax.ShapeDtypeStruct((PBUF, HIDDEN), jnp.float32),
        compiler_params=pltpu.CompilerParams(
            dimension_semantics=("arbitrary",),
        ),
        interpret=interpret,
    )(poff, x_pad, gate_up_proj, down_proj)


def _routing(top_k_index, top_k_weights):
    e_flat = top_k_index.astype(jnp.int32).reshape(-1)
    pair_id = jnp.arange(PAIRS, dtype=jnp.int32)
    e_sorted, order = lax.sort((e_flat, pair_id), num_keys=1)
    bounds = jnp.arange(NUM_EXPERTS + 1, dtype=jnp.int32)
    off = (e_flat[None, :] < bounds[:, None]).sum(axis=1).astype(jnp.int32)
    counts = off[1:] - off[:-1]
    cnt_pad = (counts + ALIGN - 1) // ALIGN * ALIGN
    poff = jnp.concatenate(
        [jnp.zeros((1,), jnp.int32), jnp.cumsum(cnt_pad)]).astype(jnp.int32)
    oh = (e_sorted[:, None]
          == jnp.arange(NUM_EXPERTS, dtype=jnp.int32)[None, :]).astype(
              jnp.float32)
    tbl = jnp.stack([off[:NUM_EXPERTS], poff[:NUM_EXPERTS]],
                    axis=1).astype(jnp.float32)
    og = jnp.dot(oh, tbl, precision=lax.Precision.HIGHEST,
                 preferred_element_type=jnp.float32).astype(jnp.int32)
    ppos = og[:, 1] + (pair_id - og[:, 0])
    tok_sorted = order // TOP_K
    tok3 = tok_sorted.reshape(SC_WORKERS, A_NCHUNK, A_CHUNK)
    pp3 = ppos.reshape(SC_WORKERS, A_NCHUNK, A_CHUNK)
    _, inv = lax.sort((order, ppos), num_keys=1)
    inv = inv.reshape(TOKENS, TOP_K)
    return tok3, pp3, poff, inv[:, 0], inv[:, 1]


def kernel(hidden_states, top_k_index, top_k_weights, gate_up_proj, down_proj):
    tok3, pp3, poff, i0, i1 = _routing(top_k_index, top_k_weights)
    w = top_k_weights.astype(jnp.float32)
    sc_dispatch, sc_combine = _sc_kernels()
    x_pad = sc_dispatch(hidden_states.astype(jnp.float32), tok3, pp3)
    y_pad = _tc_moe(x_pad, gate_up_proj, down_proj, poff)
    return sc_combine(y_pad, i0, i1, w[:, 0], w[:, 1])


# speedup vs baseline: 1.0850x; 1.0850x over previous
import functools

import jax
import jax.numpy as jnp
from jax import lax
from jax.experimental import pallas as pl
from jax.experimental.pallas import tpu as pltpu
from jax.experimental.pallas import tpu_sc as plsc

NUM_EXPERTS = 64
HIDDEN = 1024
INTER = 768
TOKENS = 2048
TOP_K = 2
PAIRS = TOKENS * TOP_K
ALIGN = 8
RT = 128
PBUF = 4672

SC_CORES = 2
SC_SUBCORES = 16
SC_WORKERS = SC_CORES * SC_SUBCORES

A_PER_W = PAIRS // SC_WORKERS
A_CHUNK = 32
A_NCHUNK = A_PER_W // A_CHUNK

C_PER_W = TOKENS // SC_WORKERS
C_CHUNK = 16
C_NCHUNK = C_PER_W // C_CHUNK

def _worker_id():
    return lax.axis_index("s") * SC_CORES + lax.axis_index("c")


@functools.lru_cache(maxsize=None)
def _sc_kernels():
    mesh = plsc.VectorSubcoreMesh(core_axis_name="c", subcore_axis_name="s")

    @functools.partial(
        pl.kernel,
        mesh=mesh,
        out_type=jax.ShapeDtypeStruct((PBUF, HIDDEN), jnp.float32),
        scratch_types=[
            pltpu.VMEM((A_NCHUNK, A_CHUNK), jnp.int32),
            pltpu.VMEM((A_NCHUNK, A_CHUNK), jnp.int32),
            pltpu.VMEM((A_CHUNK, HIDDEN), jnp.float32),
            pltpu.VMEM((A_CHUNK, HIDDEN), jnp.float32),
            pltpu.SemaphoreType.DMA,
            pltpu.SemaphoreType.DMA,
            pltpu.SemaphoreType.DMA,
            pltpu.SemaphoreType.DMA,
        ],
    )
    def sc_dispatch(hidden_hbm, tok3_hbm, pp3_hbm, out_hbm, tok_v, pp_v,
                    row0_v, row1_v, sem_g0, sem_g1, sem_s0, sem_s1):
        wid = _worker_id()
        pltpu.sync_copy(tok3_hbm.at[wid], tok_v)
        pltpu.sync_copy(pp3_hbm.at[wid], pp_v)
        bufs = (row0_v, row1_v)
        gsems = (sem_g0, sem_g1)
        ssems = (sem_s0, sem_s1)
        gathers = [None] * A_NCHUNK
        stores = [None] * A_NCHUNK
        for c in range(A_NCHUNK):
            if c >= 2:
                stores[c - 2].wait()
            gathers[c] = pltpu.async_copy(
                hidden_hbm.at[tok_v.at[c]], bufs[c % 2], gsems[c % 2])
            if c >= 1:
                gathers[c - 1].wait()
                stores[c - 1] = pltpu.async_copy(
                    bufs[(c - 1) % 2], out_hbm.at[pp_v.at[c - 1]],
                    ssems[(c - 1) % 2])
        gathers[A_NCHUNK - 1].wait()
        stores[A_NCHUNK - 1] = pltpu.async_copy(
            bufs[(A_NCHUNK - 1) % 2], out_hbm.at[pp_v.at[A_NCHUNK - 1]],
            ssems[(A_NCHUNK - 1) % 2])
        stores[A_NCHUNK - 2].wait()
        stores[A_NCHUNK - 1].wait()

    @functools.partial(
        pl.kernel,
        mesh=mesh,
        out_type=jax.ShapeDtypeStruct((TOKENS, HIDDEN), jnp.float32),
        scratch_types=[
            pltpu.VMEM((C_PER_W,), jnp.int32),
            pltpu.VMEM((C_PER_W,), jnp.int32),
            pltpu.VMEM((C_PER_W,), jnp.float32),
            pltpu.VMEM((C_PER_W,), jnp.float32),
            pltpu.VMEM((C_CHUNK, HIDDEN), jnp.float32),
            pltpu.VMEM((C_CHUNK, HIDDEN), jnp.float32),
            pltpu.VMEM((C_CHUNK, HIDDEN), jnp.float32),
            pltpu.VMEM((C_CHUNK, HIDDEN), jnp.float32),
            pltpu.VMEM((C_CHUNK, HIDDEN), jnp.float32),
            pltpu.VMEM((C_CHUNK, HIDDEN), jnp.float32),
            pltpu.SemaphoreType.DMA,
            pltpu.SemaphoreType.DMA,
            pltpu.SemaphoreType.DMA,
            pltpu.SemaphoreType.DMA,
            pltpu.SemaphoreType.DMA,
            pltpu.SemaphoreType.DMA,
            pltpu.SemaphoreType.DMA,
            pltpu.SemaphoreType.DMA,
            pltpu.SemaphoreType.DMA,
        ],
    )
    def sc_combine(ypad_hbm, i0_hbm, i1_hbm, w0_hbm, w1_hbm, out_hbm,
                   ia_v, ib_v, wa_v, wb_v, a0_v, a1_v, a2_v, b0_v, b1_v, b2_v,
                   sem_ga0, sem_ga1, sem_ga2, sem_gb0, sem_gb1, sem_gb2,
                   sem_s0, sem_s1, sem_s2):
        wid = _worker_id()
        base = wid * C_PER_W
        pltpu.sync_copy(i0_hbm.at[pl.ds(base, C_PER_W)], ia_v)
        pltpu.sync_copy(i1_hbm.at[pl.ds(base, C_PER_W)], ib_v)
        pltpu.sync_copy(w0_hbm.at[pl.ds(base, C_PER_W)], wa_v)
        pltpu.sync_copy(w1_hbm.at[pl.ds(base, C_PER_W)], wb_v)
        abufs = (a0_v, a1_v, a2_v)
        bbufs = (b0_v, b1_v, b2_v)
        ga_sems = (sem_ga0, sem_ga1, sem_ga2)
        gb_sems = (sem_gb0, sem_gb1, sem_gb2)
        s_sems = (sem_s0, sem_s1, sem_s2)
        ga = [None] * C_NCHUNK
        gb = [None] * C_NCHUNK
        st = [None] * C_NCHUNK

        def fire(c):
            ga[c] = pltpu.async_copy(
                ypad_hbm.at[ia_v.at[pl.ds(c * C_CHUNK, C_CHUNK)]],
                abufs[c % 3], ga_sems[c % 3])
            gb[c] = pltpu.async_copy(
                ypad_hbm.at[ib_v.at[pl.ds(c * C_CHUNK, C_CHUNK)]],
                bbufs[c % 3], gb_sems[c % 3])

        fire(0)
        fire(1)
        fire(2)
        for c in range(C_NCHUNK):
            ga[c].wait()
            gb[c].wait()
            a_v = abufs[c % 3]
            b_v = bbufs[c % 3]
            for r in range(C_CHUNK):
                g = c * C_CHUNK + r
                wa = wa_v[pl.ds((g // 16) * 16, 16)][g % 16]
                wb = wb_v[pl.ds((g // 16) * 16, 16)][g % 16]

                def col_body(j, _, r=r, wa=wa, wb=wb, a_v=a_v, b_v=b_v):
                    sl = pl.ds(j * 16, 16)
                    a_v[r, sl] = a_v[r, sl] * wa + b_v[r, sl] * wb
                    return 0
                lax.fori_loop(0, HIDDEN // 16, col_body, 0, unroll=8)

            st[c] = pltpu.async_copy(
                a_v, out_hbm.at[pl.ds(base + c * C_CHUNK, C_CHUNK)],
                s_sems[c % 3])
            if c + 3 < C_NCHUNK:
                st[c].wait()
                fire(c + 3)
        for c in range(max(0, C_NCHUNK - 3), C_NCHUNK):
            st[c].wait()

    return sc_dispatch, sc_combine


def _tc_moe_body(poff_ref, x_ref, gu_ref, dn_ref, y_ref):
    e = pl.program_id(0)
    start = poff_ref[e]
    end = poff_ref[e + 1]
    ntiles = (end - start + RT - 1) // RT
    dn = dn_ref[0]

    def tile(i, _):
        r0 = pl.multiple_of(start + i * RT, ALIGN)
        x = x_ref[pl.ds(r0, RT), :]
        g = lax.dot_general(x, gu_ref[0, :INTER, :],
                            (((1,), (1,)), ((), ())),
                            preferred_element_type=jnp.float32)
        u = lax.dot_general(x, gu_ref[0, INTER:, :],
                            (((1,), (1,)), ((), ())),
                            preferred_element_type=jnp.float32)
        h = g * jax.nn.sigmoid(g) * u
        y = lax.dot_general(h, dn, (((1,), (1,)), ((), ())),
                            preferred_element_type=jnp.float32)
        y_ref[pl.ds(r0, RT), :] = y
        return 0

    lax.fori_loop(0, ntiles, tile, 0)


def _tc_moe(x_pad, gate_up_proj, down_proj, poff, interpret=False):
    grid_spec = pltpu.PrefetchScalarGridSpec(
        num_scalar_prefetch=1,
        grid=(NUM_EXPERTS,),
        in_specs=[
            pl.BlockSpec((PBUF, HIDDEN), lambda e, poff: (0, 0)),
            pl.BlockSpec((1, 2 * INTER, HIDDEN), lambda e, poff: (e, 0, 0)),
            pl.BlockSpec((1, HIDDEN, INTER), lambda e, poff: (e, 0, 0)),
        ],
        out_specs=pl.BlockSpec((PBUF, HIDDEN), lambda e, poff: (0, 0)),
    )
    return pl.pallas_call(
        _tc_moe_body,
        grid_spec=grid_spec,
        out_shape=jax.ShapeDtypeStruct((PBUF, HIDDEN), jnp.float32),
        compiler_params=pltpu.CompilerParams(
            dimension_semantics=("arbitrary",),
        ),
        interpret=interpret,
    )(poff, x_pad, gate_up_proj, down_proj)


def _routing(top_k_index, top_k_weights):
    e_flat = top_k_index.astype(jnp.int32).reshape(-1)
    pair_id = jnp.arange(PAIRS, dtype=jnp.int32)
    e_sorted, order = lax.sort((e_flat, pair_id), num_keys=1)
    bounds = jnp.arange(NUM_EXPERTS + 1, dtype=jnp.int32)
    off = (e_flat[None, :] < bounds[:, None]).sum(axis=1).astype(jnp.int32)
    counts = off[1:] - off[:-1]
    cnt_pad = (counts + ALIGN - 1) // ALIGN * ALIGN
    poff = jnp.concatenate(
        [jnp.zeros((1,), jnp.int32), jnp.cumsum(cnt_pad)]).astype(jnp.int32)
    oh = (e_sorted[:, None]
          == jnp.arange(NUM_EXPERTS, dtype=jnp.int32)[None, :]).astype(
              jnp.float32)
    tbl = jnp.stack([off[:NUM_EXPERTS], poff[:NUM_EXPERTS]],
                    axis=1).astype(jnp.float32)
    og = jnp.dot(oh, tbl, precision=lax.Precision.HIGHEST,
                 preferred_element_type=jnp.float32).astype(jnp.int32)
    ppos = og[:, 1] + (pair_id - og[:, 0])
    tok_sorted = order // TOP_K
    tok3 = tok_sorted.reshape(SC_WORKERS, A_NCHUNK, A_CHUNK)
    pp3 = ppos.reshape(SC_WORKERS, A_NCHUNK, A_CHUNK)
    _, inv = lax.sort((order, ppos), num_keys=1)
    inv = inv.reshape(TOKENS, TOP_K)
    return tok3, pp3, poff, inv[:, 0], inv[:, 1]


def kernel(hidden_states, top_k_index, top_k_weights, gate_up_proj, down_proj):
    tok3, pp3, poff, i0, i1 = _routing(top_k_index, top_k_weights)
    w = top_k_weights.astype(jnp.float32)
    sc_dispatch, sc_combine = _sc_kernels()
    x_pad = sc_dispatch(hidden_states.astype(jnp.float32), tok3, pp3)
    y_pad = _tc_moe(x_pad, gate_up_proj, down_proj, poff)
    return sc_combine(y_pad, i0, i1, w[:, 0], w[:, 1])
